# Initial kernel scaffold; baseline (speedup 1.0000x reference)
import jax
import jax.numpy as jnp
from jax.experimental import pallas as pl


def kernel(x, x_0, edge_index, W, lin_w, bn_gamma, bn_beta):
    raise NotImplementedError("write your pallas kernel here")


# speedup vs baseline: 3.7266x; 3.7266x over previous
import functools

import numpy as np
import jax
import jax.numpy as jnp
from jax import lax
from jax.experimental import pallas as pl
from jax.experimental.pallas import tpu as pltpu
from jax.experimental.pallas import tpu_sc as plsc

N = 10000
E = 320000
D = 128
ALPHA = 0.1
BETA = float(np.log(0.5 / 1 + 1.0))

NC = 2
NS = 16
NW = NC * NS
B = 128
KPT = 158
NBATCH = KPT * NS
E_PAD = NBATCH * B
NPAD = 10240
NH = N // NC
AGG = 5120
DH = D // NC
EPW = E // NW

_mesh = plsc.VectorSubcoreMesh(core_axis_name="c", subcore_axis_name="s")


@functools.partial(
    pl.kernel,
    mesh=_mesh,
    out_type=jax.ShapeDtypeStruct((NW, N), jnp.float32),
    scratch_types=[
        pltpu.VMEM((EPW,), jnp.int32),
        pltpu.VMEM((N,), jnp.float32),
    ],
    compiler_params=pltpu.CompilerParams(needs_layout_passes=False),
)
def _deg_kernel(dst_hbm, deg_hbm, dst_v, cnt_v):
    cid = lax.axis_index("c")
    sid = lax.axis_index("s")
    wid = cid * NS + sid
    pltpu.sync_copy(dst_hbm.at[wid], dst_v)
    zeros = jnp.zeros((16,), jnp.float32)
    ones = jnp.ones((16,), jnp.float32)

    def zero_body(i, carry):
        cnt_v[pl.ds(i * 16, 16)] = zeros
        return carry

    lax.fori_loop(0, N // 16, zero_body, 0)

    def scat_body(g, carry):
        idx = dst_v[pl.ds(g * 16, 16)]
        plsc.addupdate_scatter(cnt_v, [idx], ones)
        return carry

    lax.fori_loop(0, EPW // 16, scat_body, 0)
    pltpu.sync_copy(cnt_v, deg_hbm.at[wid])


def _tc1_body(parts_ref, x_ref, y_ref, ab_ref, dinv_ref):
    deg = jnp.sum(parts_ref[...], axis=0) + 1.0
    dinv = lax.rsqrt(deg)
    x = x_ref[...]
    y_ref[0:N, :] = x * dinv[:, None]
    y_ref[N:NPAD, :] = jnp.zeros((NPAD - N, D), jnp.float32)
    a = jnp.sqrt(deg)
    b = jnp.sqrt(jnp.sum(x * x, axis=1))
    ab_ref[...] = jnp.stack([a, b])
    dinv_ref[...] = dinv[:, None]


_tc1 = pl.pallas_call(
    _tc1_body,
    out_shape=(
        jax.ShapeDtypeStruct((NPAD, D), jnp.float32),
        jax.ShapeDtypeStruct((2, N), jnp.float32),
        jax.ShapeDtypeStruct((N, 1), jnp.float32),
    ),
)


@functools.partial(
    pl.kernel,
    mesh=_mesh,
    out_type=(
        jax.ShapeDtypeStruct((NC * AGG, D), jnp.float32),
        jax.ShapeDtypeStruct((NW, 16), jnp.float32),
    ),
    scratch_types=[
        pltpu.VMEM((N,), jnp.float32),
        pltpu.VMEM((N,), jnp.float32),
        pltpu.VMEM((B,), jnp.int32),
        pltpu.VMEM((B,), jnp.int32),
        pltpu.VMEM((B,), jnp.int32),
        pltpu.VMEM((B, D), jnp.float32),
        pltpu.VMEM((B, D), jnp.float32),
        pltpu.VMEM((16,), jnp.float32),
        pltpu.VMEM_SHARED((AGG, D), jnp.float32),
        pltpu.SemaphoreType.DMA,
        pltpu.SemaphoreType.DMA,
    ],
    compiler_params=pltpu.CompilerParams(needs_layout_passes=False),
)
def _edge_kernel(y_hbm, src_hbm, dst_hbm, ab_hbm, pout_hbm, cos_hbm,
                 a_v, b_v, si_v, di_v, dio_v, ys_v, yd_v, ca_v,
                 agg_sh, sem1, sem2):
    cid = lax.axis_index("c")
    sid = lax.axis_index("s")
    wid = cid * NS + sid
    pltpu.sync_copy(ab_hbm.at[0], a_v)
    pltpu.sync_copy(ab_hbm.at[1], b_v)

    zeros = jnp.zeros((16,), jnp.float32)

    def zb(i, carry):
        r = i // (D // 16)
        c = i % (D // 16)
        ys_v[r, pl.ds(c * 16, 16)] = zeros
        return carry

    lax.fori_loop(0, B * (D // 16), zb, 0)

    rpt = AGG // NS
    zc = 64

    def za(t, carry):
        pltpu.sync_copy(ys_v.at[pl.ds(0, zc)],
                        agg_sh.at[pl.ds(sid * rpt + t * zc, zc)])
        return carry

    lax.fori_loop(0, rpt // zc, za, 0)
    plsc.subcore_barrier()

    iota = lax.iota(jnp.int32, 16)
    nbase = cid * NH

    def batch_body(k, acc):
        j = k * NS + sid
        pltpu.sync_copy(src_hbm.at[pl.ds(j * B, B)], si_v)
        pltpu.sync_copy(dst_hbm.at[pl.ds(j * B, B)], di_v)
        cp1 = pltpu.async_copy(y_hbm.at[si_v], ys_v, sem1)
        cp2 = pltpu.async_copy(y_hbm.at[di_v], yd_v, sem2)
        cp1.wait()
        cp2.wait()
        for g in range(B // 16):
            sl = pl.ds(g * 16, 16)
            e16 = iota + (g * 16)
            s16 = si_v[sl]
            d16 = di_v[sl]
            dloc = d16 - nbase
            mine = (dloc >= 0) & (dloc < NH)
            dio_v[sl] = jnp.where(mine, dloc, NH)
            a_s = plsc.load_gather(a_v, [s16])
            a_d = plsc.load_gather(a_v, [d16])
            b_s = plsc.load_gather(b_v, [s16])
            b_d = plsc.load_gather(b_v, [d16])
            c16 = (a_s * a_d) / (b_s * b_d + 1e-8)

            def dot_body(t, dacc):
                d0 = cid * DH + t * 8
                for dd in range(8):
                    dv = jnp.full((16,), d0 + dd, jnp.int32)
                    v1 = plsc.load_gather(ys_v, [e16, dv])
                    v2 = plsc.load_gather(yd_v, [e16, dv])
                    dacc = dacc + v1 * v2
                return dacc

            dot = lax.fori_loop(0, DH // 8, dot_body, zeros)
            eglob = j * B + g * 16 + iota
            acc = acc + jnp.where(eglob < E, dot * c16, 0.0)
        pltpu.sync_copy(ys_v, agg_sh.at[dio_v], add=True)
        return acc

    acc = lax.fori_loop(0, KPT, batch_body, zeros)
    ca_v[...] = acc
    pltpu.sync_copy(ca_v, cos_hbm.at[wid])
    plsc.subcore_barrier()

    def wb(t, carry):
        row = sid * rpt + t * zc
        pltpu.sync_copy(agg_sh.at[pl.ds(row, zc)],
                        pout_hbm.at[pl.ds(cid * AGG + row, zc)])
        return carry

    lax.fori_loop(0, rpt // zc, wb, 0)


def _tc2_body(p_ref, x_ref, x0_ref, dinv_ref, w_ref, lw_ref, g_ref, bb_ref,
              cos_ref, out_ref, gcd_ref):
    dinv = dinv_ref[...]
    p = jnp.concatenate(
        [p_ref[0:NH, :], p_ref[AGG:AGG + NH, :]], axis=0)
    x = x_ref[...]
    agg = dinv * p + (dinv * dinv) * x
    h = (1.0 - ALPHA) * agg + ALPHA * x0_ref[...]
    hw = lax.dot_general(h, w_ref[...], (((1,), (1,)), ((), ())),
                         preferred_element_type=jnp.float32)
    out1 = (1.0 - BETA) * h + BETA * hw
    mu = jnp.mean(out1, axis=0)
    cen = out1 - mu[None, :]
    var = jnp.mean(cen * cen, axis=0)
    o = cen * lax.rsqrt(var + 1e-5)[None, :] * g_ref[...] + bb_ref[...]
    o = jnp.maximum(o, 0.0)
    xl = lax.dot_general(x, lw_ref[...], (((1,), (1,)), ((), ())),
                         preferred_element_type=jnp.float32)
    out_ref[...] = o + xl
    gcd_ref[...] = jnp.reshape(1.0 - jnp.sum(cos_ref[...]) * (1.0 / E), (1, 1))


_tc2 = pl.pallas_call(
    _tc2_body,
    out_shape=(
        jax.ShapeDtypeStruct((N, D), jnp.float32),
        jax.ShapeDtypeStruct((1, 1), jnp.float32),
    ),
)


def kernel(x, x_0, edge_index, W, lin_w, bn_gamma, bn_beta):
    src = edge_index[0].astype(jnp.int32)
    dst = edge_index[1].astype(jnp.int32)
    pad = E_PAD - E
    src_p = jnp.concatenate([src, jnp.zeros((pad,), jnp.int32)])
    dst_p = jnp.concatenate(
        [dst, N + (jnp.arange(pad, dtype=jnp.int32) % (NPAD - N))])

    deg_parts = _deg_kernel(dst.reshape(NW, EPW))
    y, ab, dinv = _tc1(deg_parts, x)
    pout, cosp = _edge_kernel(y, src_p, dst_p, ab)
    out, gcd = _tc2(pout, x, x_0, dinv, W, lin_w,
                    bn_gamma.reshape(1, D), bn_beta.reshape(1, D), cosp)
    return out, gcd.reshape(())


# speedup vs baseline: 4.6711x; 1.2534x over previous
import functools

import numpy as np
import jax
import jax.numpy as jnp
from jax import lax
from jax.experimental import pallas as pl
from jax.experimental.pallas import tpu as pltpu
from jax.experimental.pallas import tpu_sc as plsc

N = 10000
E = 320000
D = 128
ALPHA = 0.1
BETA = float(np.log(0.5 / 1 + 1.0))

NC = 2
NS = 16
NW = NC * NS
B = 128
KPT = 158
NBATCH = KPT * NS
E_PAD = NBATCH * B
NPAD = 10240
NH = N // NC
AGG = 5120
DH = D // NC
EPW = E // NW

_mesh = plsc.VectorSubcoreMesh(core_axis_name="c", subcore_axis_name="s")


@functools.partial(
    pl.kernel,
    mesh=_mesh,
    out_type=jax.ShapeDtypeStruct((NW, N), jnp.float32),
    scratch_types=[
        pltpu.VMEM((EPW,), jnp.int32),
        pltpu.VMEM((N,), jnp.float32),
    ],
    compiler_params=pltpu.CompilerParams(needs_layout_passes=False),
)
def _deg_kernel(dst_hbm, deg_hbm, dst_v, cnt_v):
    cid = lax.axis_index("c")
    sid = lax.axis_index("s")
    wid = cid * NS + sid
    pltpu.sync_copy(dst_hbm.at[wid], dst_v)
    zeros = jnp.zeros((16,), jnp.float32)
    ones = jnp.ones((16,), jnp.float32)

    def zero_body(i, carry):
        cnt_v[pl.ds(i * 16, 16)] = zeros
        return carry

    lax.fori_loop(0, N // 16, zero_body, 0)

    def scat_body(g, carry):
        idx = dst_v[pl.ds(g * 16, 16)]
        plsc.addupdate_scatter(cnt_v, [idx], ones)
        return carry

    lax.fori_loop(0, EPW // 16, scat_body, 0)
    pltpu.sync_copy(cnt_v, deg_hbm.at[wid])


def _tc1_body(parts_ref, x_ref, y_ref, ab_ref, dinv_ref):
    deg = jnp.sum(parts_ref[...], axis=0) + 1.0
    dinv = lax.rsqrt(deg)
    x = x_ref[...]
    y_ref[0:N, :] = x * dinv[:, None]
    y_ref[N:NPAD, :] = jnp.zeros((NPAD - N, D), jnp.float32)
    a = jnp.sqrt(deg)
    b = jnp.sqrt(jnp.sum(x * x, axis=1))
    ab_ref[...] = jnp.stack([a, b])
    dinv_ref[...] = dinv[:, None]


_tc1 = pl.pallas_call(
    _tc1_body,
    out_shape=(
        jax.ShapeDtypeStruct((NPAD, D), jnp.float32),
        jax.ShapeDtypeStruct((2, N), jnp.float32),
        jax.ShapeDtypeStruct((N, 1), jnp.float32),
    ),
)


@functools.partial(
    pl.kernel,
    mesh=_mesh,
    out_type=(
        jax.ShapeDtypeStruct((NC * AGG, D), jnp.float32),
        jax.ShapeDtypeStruct((NW, 16), jnp.float32),
    ),
    scratch_types=[
        pltpu.VMEM((N,), jnp.float32),
        pltpu.VMEM((N,), jnp.float32),
        pltpu.VMEM((2, 2, B), jnp.int32),
        pltpu.VMEM((2, B), jnp.int32),
        pltpu.VMEM((2, B, D), jnp.float32),
        pltpu.VMEM((2, B, D), jnp.float32),
        pltpu.VMEM((16,), jnp.float32),
        pltpu.VMEM_SHARED((AGG, D), jnp.float32),
        pltpu.SemaphoreType.DMA,
        pltpu.SemaphoreType.DMA,
        pltpu.SemaphoreType.DMA,
        pltpu.SemaphoreType.DMA,
        pltpu.SemaphoreType.DMA,
        pltpu.SemaphoreType.DMA,
    ],
    compiler_params=pltpu.CompilerParams(needs_layout_passes=False),
)
def _edge_kernel(y_hbm, ei_hbm, ab_hbm, pout_hbm, cos_hbm,
                 a_v, b_v, se_v, dio_v, ys_v, yd_v, ca_v,
                 agg_sh, sg0, sg1, sh0, sh1, ss0, ss1):
    cid = lax.axis_index("c")
    sid = lax.axis_index("s")
    wid = cid * NS + sid
    sg = (sg0, sg1)
    sh = (sh0, sh1)
    ss = (ss0, ss1)
    pltpu.sync_copy(ab_hbm.at[0], a_v)
    pltpu.sync_copy(ab_hbm.at[1], b_v)

    zeros = jnp.zeros((16,), jnp.float32)

    def zb(i, carry):
        r = i // (D // 16)
        c = i % (D // 16)
        ys_v[0, r, pl.ds(c * 16, 16)] = zeros
        return carry

    lax.fori_loop(0, B * (D // 16), zb, 0)

    rpt = AGG // NS
    zc = 64

    def za(t, carry):
        pltpu.sync_copy(ys_v.at[0, pl.ds(0, zc)],
                        agg_sh.at[pl.ds(sid * rpt + t * zc, zc)])
        return carry

    lax.fori_loop(0, rpt // zc, za, 0)
    plsc.subcore_barrier()

    iota = lax.iota(jnp.int32, 16)
    nbase = cid * NH

    def idx_load(k, slot):
        pltpu.sync_copy(ei_hbm.at[k * NS + sid], se_v.at[slot])

    def gather_start(slot):
        pltpu.async_copy(y_hbm.at[se_v.at[slot, 0]], ys_v.at[slot], sg[slot])
        pltpu.async_copy(y_hbm.at[se_v.at[slot, 1]], yd_v.at[slot], sh[slot])

    def gather_wait(slot):
        pltpu.make_async_copy(
            y_hbm.at[se_v.at[slot, 0]], ys_v.at[slot], sg[slot]).wait()
        pltpu.make_async_copy(
            y_hbm.at[se_v.at[slot, 1]], yd_v.at[slot], sh[slot]).wait()

    def scatter_start(slot):
        pltpu.async_copy(ys_v.at[slot], agg_sh.at[dio_v.at[slot]],
                         ss[slot], add=True)

    def scatter_wait(slot):
        pltpu.make_async_copy(
            ys_v.at[slot], agg_sh.at[dio_v.at[slot]], ss[slot]).wait()

    idx_load(0, 0)
    gather_start(0)

    def pair_body(i, acc):
        for b in range(2):
            k = i * 2 + b
            nxt = 1 - b

            @pl.when(k < KPT - 1)
            def _():
                idx_load(k + 1, nxt)

            @pl.when(jnp.logical_and(k > 0, k < KPT - 1))
            def _():
                scatter_wait(nxt)

            @pl.when(k < KPT - 1)
            def _():
                gather_start(nxt)

            gather_wait(b)
            for g in range(B // 16):
                sl = pl.ds(g * 16, 16)
                e16 = iota + (g * 16)
                s16 = se_v[b, 0, sl]
                d16 = se_v[b, 1, sl]
                dloc = d16 - nbase
                mine = (dloc >= 0) & (dloc < NH)
                dio_v[b, sl] = jnp.where(mine, dloc, NH)
                a_s = plsc.load_gather(a_v, [s16])
                a_d = plsc.load_gather(a_v, [d16])
                b_s = plsc.load_gather(b_v, [s16])
                b_d = plsc.load_gather(b_v, [d16])
                c16 = (a_s * a_d) / (b_s * b_d + 1e-8)

                def dot_body(t, dacc):
                    d0 = cid * DH + t * 16
                    for dd in range(16):
                        dv = jnp.full((16,), d0 + dd, jnp.int32)
                        v1 = plsc.load_gather(ys_v.at[b], [e16, dv])
                        v2 = plsc.load_gather(yd_v.at[b], [e16, dv])
                        dacc = dacc + v1 * v2
                    return dacc

                dot = lax.fori_loop(0, DH // 16, dot_body, zeros)
                eglob = (k * NS + sid) * B + g * 16 + iota
                acc = acc + jnp.where(eglob < E, dot * c16, 0.0)
            scatter_start(b)
        return acc

    acc = lax.fori_loop(0, KPT // 2, pair_body, zeros)
    scatter_wait(0)
    scatter_wait(1)
    ca_v[...] = acc
    pltpu.sync_copy(ca_v, cos_hbm.at[wid])
    plsc.subcore_barrier()

    def wb(t, carry):
        row = sid * rpt + t * zc
        pltpu.sync_copy(agg_sh.at[pl.ds(row, zc)],
                        pout_hbm.at[pl.ds(cid * AGG + row, zc)])
        return carry

    lax.fori_loop(0, rpt // zc, wb, 0)


def _tc2_body(p_ref, x_ref, x0_ref, dinv_ref, w_ref, lw_ref, g_ref, bb_ref,
              cos_ref, out_ref, gcd_ref):
    dinv = dinv_ref[...]
    p = jnp.concatenate(
        [p_ref[0:NH, :], p_ref[AGG:AGG + NH, :]], axis=0)
    x = x_ref[...]
    agg = dinv * p + (dinv * dinv) * x
    h = (1.0 - ALPHA) * agg + ALPHA * x0_ref[...]
    hw = lax.dot_general(h, w_ref[...], (((1,), (1,)), ((), ())),
                         preferred_element_type=jnp.float32)
    out1 = (1.0 - BETA) * h + BETA * hw
    mu = jnp.mean(out1, axis=0)
    cen = out1 - mu[None, :]
    var = jnp.mean(cen * cen, axis=0)
    o = cen * lax.rsqrt(var + 1e-5)[None, :] * g_ref[...] + bb_ref[...]
    o = jnp.maximum(o, 0.0)
    xl = lax.dot_general(x, lw_ref[...], (((1,), (1,)), ((), ())),
                         preferred_element_type=jnp.float32)
    out_ref[...] = o + xl
    gcd_ref[...] = jnp.reshape(1.0 - jnp.sum(cos_ref[...]) * (1.0 / E), (1, 1))


_tc2 = pl.pallas_call(
    _tc2_body,
    out_shape=(
        jax.ShapeDtypeStruct((N, D), jnp.float32),
        jax.ShapeDtypeStruct((1, 1), jnp.float32),
    ),
)


def kernel(x, x_0, edge_index, W, lin_w, bn_gamma, bn_beta):
    src = edge_index[0].astype(jnp.int32)
    dst = edge_index[1].astype(jnp.int32)
    pad = E_PAD - E
    src_p = jnp.concatenate([src, jnp.zeros((pad,), jnp.int32)])
    dst_p = jnp.concatenate(
        [dst, N + (jnp.arange(pad, dtype=jnp.int32) % (NPAD - N))])

    edges_p = jnp.stack(
        [src_p.reshape(NBATCH, B), dst_p.reshape(NBATCH, B)], axis=1)

    deg_parts = _deg_kernel(dst.reshape(NW, EPW))
    y, ab, dinv = _tc1(deg_parts, x)
    pout, cosp = _edge_kernel(y, edges_p, ab)
    out, gcd = _tc2(pout, x, x_0, dinv, W, lin_w,
                    bn_gamma.reshape(1, D), bn_beta.reshape(1, D), cosp)
    return out, gcd.reshape(())


# speedup vs baseline: 11.2059x; 2.3990x over previous
import functools

import numpy as np
import jax
import jax.numpy as jnp
from jax import lax
from jax.experimental import pallas as pl
from jax.experimental.pallas import tpu as pltpu
from jax.experimental.pallas import tpu_sc as plsc

N = 10000
E = 320000
D = 128
ALPHA = 0.1
BETA = float(np.log(0.5 / 1 + 1.0))

NC = 2
NS = 16
NW = NC * NS
B = 128
KPT = 158
NBATCH = KPT * NS
E_PAD = NBATCH * B
NPAD = 10240
NH = N // NC
AGG = 5120
DH = D // NC
EPW = E // NW

_mesh = plsc.VectorSubcoreMesh(core_axis_name="c", subcore_axis_name="s")


@functools.partial(
    pl.kernel,
    mesh=_mesh,
    out_type=jax.ShapeDtypeStruct((NW, N), jnp.float32),
    scratch_types=[
        pltpu.VMEM((EPW,), jnp.int32),
        pltpu.VMEM((N,), jnp.float32),
    ],
    compiler_params=pltpu.CompilerParams(needs_layout_passes=False),
)
def _deg_kernel(dst_hbm, deg_hbm, dst_v, cnt_v):
    cid = lax.axis_index("c")
    sid = lax.axis_index("s")
    wid = cid * NS + sid
    pltpu.sync_copy(dst_hbm.at[wid], dst_v)
    zeros = jnp.zeros((16,), jnp.float32)
    ones = jnp.ones((16,), jnp.float32)

    def zero_body(i, carry):
        cnt_v[pl.ds(i * 16, 16)] = zeros
        return carry

    lax.fori_loop(0, N // 16, zero_body, 0)

    def scat_body(g, carry):
        idx = dst_v[pl.ds(g * 16, 16)]
        plsc.addupdate_scatter(cnt_v, [idx], ones)
        return carry

    lax.fori_loop(0, EPW // 16, scat_body, 0)
    pltpu.sync_copy(cnt_v, deg_hbm.at[wid])


def _tc1_body(parts_ref, x_ref, y_ref, ab_ref, dinv_ref):
    deg = jnp.sum(parts_ref[...], axis=0) + 1.0
    dinv = lax.rsqrt(deg)
    x = x_ref[...]
    y_ref[0:N, :] = x * dinv[:, None]
    y_ref[N:NPAD, :] = jnp.zeros((NPAD - N, D), jnp.float32)
    a = jnp.sqrt(deg)
    b = jnp.sqrt(jnp.sum(x * x, axis=1))
    ab_ref[...] = jnp.stack([a, b])
    dinv_ref[...] = dinv[:, None]


_tc1 = pl.pallas_call(
    _tc1_body,
    out_shape=(
        jax.ShapeDtypeStruct((NPAD, D), jnp.float32),
        jax.ShapeDtypeStruct((2, N), jnp.float32),
        jax.ShapeDtypeStruct((N, 1), jnp.float32),
    ),
)


@functools.partial(
    pl.kernel,
    mesh=_mesh,
    out_type=(
        jax.ShapeDtypeStruct((NC * AGG, D), jnp.float32),
        jax.ShapeDtypeStruct((NW, 16), jnp.float32),
    ),
    scratch_types=[
        pltpu.VMEM((N,), jnp.float32),
        pltpu.VMEM((N,), jnp.float32),
        pltpu.VMEM((2, 2, B), jnp.int32),
        pltpu.VMEM((2, B), jnp.int32),
        pltpu.VMEM((2, B, D), jnp.float32),
        pltpu.VMEM((2, B, D), jnp.float32),
        pltpu.VMEM((16,), jnp.float32),
        pltpu.VMEM_SHARED((AGG, D), jnp.float32),
        pltpu.SemaphoreType.DMA,
        pltpu.SemaphoreType.DMA,
        pltpu.SemaphoreType.DMA,
        pltpu.SemaphoreType.DMA,
        pltpu.SemaphoreType.DMA,
        pltpu.SemaphoreType.DMA,
    ],
    compiler_params=pltpu.CompilerParams(needs_layout_passes=False),
)
def _edge_kernel(y_hbm, ei_hbm, ab_hbm, pout_hbm, cos_hbm,
                 a_v, b_v, se_v, dio_v, ys_v, yd_v, ca_v,
                 agg_sh, sg0, sg1, sh0, sh1, ss0, ss1):
    cid = lax.axis_index("c")
    sid = lax.axis_index("s")
    wid = cid * NS + sid
    sg = (sg0, sg1)
    sh = (sh0, sh1)
    ss = (ss0, ss1)
    pltpu.sync_copy(ab_hbm.at[0], a_v)
    pltpu.sync_copy(ab_hbm.at[1], b_v)

    zeros = jnp.zeros((16,), jnp.float32)

    def zb(i, carry):
        r = i // (D // 16)
        c = i % (D // 16)
        ys_v[0, r, pl.ds(c * 16, 16)] = zeros
        return carry

    lax.fori_loop(0, B * (D // 16), zb, 0)

    rpt = AGG // NS
    zc = 64

    def za(t, carry):
        pltpu.sync_copy(ys_v.at[0, pl.ds(0, zc)],
                        agg_sh.at[pl.ds(sid * rpt + t * zc, zc)])
        return carry

    lax.fori_loop(0, rpt // zc, za, 0)
    plsc.subcore_barrier()

    iota = lax.iota(jnp.int32, 16)
    nbase = cid * NH

    def idx_load(k, slot):
        pltpu.sync_copy(ei_hbm.at[k * NS + sid], se_v.at[slot])

    def gather_start(slot):
        pltpu.async_copy(y_hbm.at[se_v.at[slot, 0]], ys_v.at[slot], sg[slot])
        pltpu.async_copy(y_hbm.at[se_v.at[slot, 1]], yd_v.at[slot], sh[slot])

    def gather_wait(slot):
        pltpu.make_async_copy(
            y_hbm.at[se_v.at[slot, 0]], ys_v.at[slot], sg[slot]).wait()
        pltpu.make_async_copy(
            y_hbm.at[se_v.at[slot, 1]], yd_v.at[slot], sh[slot]).wait()

    def scatter_start(slot):
        pltpu.async_copy(ys_v.at[slot], agg_sh.at[dio_v.at[slot]],
                         ss[slot], add=True)

    def scatter_wait(slot):
        pltpu.make_async_copy(
            ys_v.at[slot], agg_sh.at[dio_v.at[slot]], ss[slot]).wait()

    idx_load(0, 0)
    gather_start(0)

    def pair_body(i, acc):
        for b in range(2):
            k = i * 2 + b
            nxt = 1 - b

            @pl.when(k < KPT - 1)
            def _():
                idx_load(k + 1, nxt)

            @pl.when(jnp.logical_and(k > 0, k < KPT - 1))
            def _():
                scatter_wait(nxt)

            @pl.when(k < KPT - 1)
            def _():
                gather_start(nxt)

            gather_wait(b)
            dbase = cid * DH
            for g in range(B // 16):
                sl = pl.ds(g * 16, 16)
                s16 = se_v[b, 0, sl]
                d16 = se_v[b, 1, sl]
                dloc = d16 - nbase
                mine = (dloc >= 0) & (dloc < NH)
                dio_v[b, sl] = jnp.where(mine, dloc, NH)
                a_s = plsc.load_gather(a_v, [s16])
                a_d = plsc.load_gather(a_v, [d16])
                b_s = plsc.load_gather(b_v, [s16])
                b_d = plsc.load_gather(b_v, [d16])
                c16 = (a_s * a_d) / (b_s * b_d + 1e-8)

                def edge_body(it, dot16):
                    for u in range(4):
                        e = g * 16 + it * 4 + u
                        ps = None
                        for q in range(DH // 16):
                            dsl = pl.ds(dbase + q * 16, 16)
                            v1 = ys_v[b, e, dsl]
                            v2 = yd_v[b, e, dsl]
                            ps = v1 * v2 if ps is None else ps + v1 * v2
                        s = jnp.sum(ps)
                        dot16 = jnp.where(iota == it * 4 + u, s, dot16)
                    return dot16

                dot = lax.fori_loop(0, 4, edge_body, zeros)
                eglob = (k * NS + sid) * B + g * 16 + iota
                acc = acc + jnp.where(eglob < E, dot * c16, 0.0)
            scatter_start(b)
        return acc

    acc = lax.fori_loop(0, KPT // 2, pair_body, zeros)
    scatter_wait(0)
    scatter_wait(1)
    ca_v[...] = acc
    pltpu.sync_copy(ca_v, cos_hbm.at[wid])
    plsc.subcore_barrier()

    def wb(t, carry):
        row = sid * rpt + t * zc
        pltpu.sync_copy(agg_sh.at[pl.ds(row, zc)],
                        pout_hbm.at[pl.ds(cid * AGG + row, zc)])
        return carry

    lax.fori_loop(0, rpt // zc, wb, 0)


def _tc2_body(p_ref, x_ref, x0_ref, dinv_ref, w_ref, lw_ref, g_ref, bb_ref,
              cos_ref, out_ref, gcd_ref):
    dinv = dinv_ref[...]
    p = jnp.concatenate(
        [p_ref[0:NH, :], p_ref[AGG:AGG + NH, :]], axis=0)
    x = x_ref[...]
    agg = dinv * p + (dinv * dinv) * x
    h = (1.0 - ALPHA) * agg + ALPHA * x0_ref[...]
    hw = lax.dot_general(h, w_ref[...], (((1,), (1,)), ((), ())),
                         preferred_element_type=jnp.float32)
    out1 = (1.0 - BETA) * h + BETA * hw
    mu = jnp.mean(out1, axis=0)
    cen = out1 - mu[None, :]
    var = jnp.mean(cen * cen, axis=0)
    o = cen * lax.rsqrt(var + 1e-5)[None, :] * g_ref[...] + bb_ref[...]
    o = jnp.maximum(o, 0.0)
    xl = lax.dot_general(x, lw_ref[...], (((1,), (1,)), ((), ())),
                         preferred_element_type=jnp.float32)
    out_ref[...] = o + xl
    gcd_ref[...] = jnp.reshape(1.0 - jnp.sum(cos_ref[...]) * (1.0 / E), (1, 1))


_tc2 = pl.pallas_call(
    _tc2_body,
    out_shape=(
        jax.ShapeDtypeStruct((N, D), jnp.float32),
        jax.ShapeDtypeStruct((1, 1), jnp.float32),
    ),
)


def kernel(x, x_0, edge_index, W, lin_w, bn_gamma, bn_beta):
    src = edge_index[0].astype(jnp.int32)
    dst = edge_index[1].astype(jnp.int32)
    pad = E_PAD - E
    src_p = jnp.concatenate([src, jnp.zeros((pad,), jnp.int32)])
    dst_p = jnp.concatenate(
        [dst, N + (jnp.arange(pad, dtype=jnp.int32) % (NPAD - N))])

    edges_p = jnp.stack(
        [src_p.reshape(NBATCH, B), dst_p.reshape(NBATCH, B)], axis=1)

    deg_parts = _deg_kernel(dst.reshape(NW, EPW))
    y, ab, dinv = _tc1(deg_parts, x)
    pout, cosp = _edge_kernel(y, edges_p, ab)
    out, gcd = _tc2(pout, x, x_0, dinv, W, lin_w,
                    bn_gamma.reshape(1, D), bn_beta.reshape(1, D), cosp)
    return out, gcd.reshape(())


# speedup vs baseline: 11.2683x; 1.0056x over previous
import functools

import numpy as np
import jax
import jax.numpy as jnp
from jax import lax
from jax.experimental import pallas as pl
from jax.experimental.pallas import tpu as pltpu
from jax.experimental.pallas import tpu_sc as plsc

N = 10000
E = 320000
D = 128
ALPHA = 0.1
BETA = float(np.log(0.5 / 1 + 1.0))

NC = 2
NS = 16
NW = NC * NS
B = 128
KPT = 158
NBATCH = KPT * NS
E_PAD = NBATCH * B
NPAD = 10240
NH = N // NC
AGG = 5120
DH = D // NC
EPW = E // NW

_mesh = plsc.VectorSubcoreMesh(core_axis_name="c", subcore_axis_name="s")


@functools.partial(
    pl.kernel,
    mesh=_mesh,
    out_type=jax.ShapeDtypeStruct((NW, N), jnp.float32),
    scratch_types=[
        pltpu.VMEM((EPW,), jnp.int32),
        pltpu.VMEM((N,), jnp.float32),
    ],
    compiler_params=pltpu.CompilerParams(needs_layout_passes=False),
)
def _deg_kernel(dst_hbm, deg_hbm, dst_v, cnt_v):
    cid = lax.axis_index("c")
    sid = lax.axis_index("s")
    wid = cid * NS + sid
    pltpu.sync_copy(dst_hbm.at[wid], dst_v)
    zeros = jnp.zeros((16,), jnp.float32)
    ones = jnp.ones((16,), jnp.float32)

    def zero_body(i, carry):
        cnt_v[pl.ds(i * 16, 16)] = zeros
        return carry

    lax.fori_loop(0, N // 16, zero_body, 0)

    def scat_body(g, carry):
        idx = dst_v[pl.ds(g * 16, 16)]
        plsc.addupdate_scatter(cnt_v, [idx], ones)
        return carry

    lax.fori_loop(0, EPW // 16, scat_body, 0)
    pltpu.sync_copy(cnt_v, deg_hbm.at[wid])


def _tc1_body(parts_ref, x_ref, y_ref, ab_ref, dinv_ref):
    deg = jnp.sum(parts_ref[...], axis=0) + 1.0
    dinv = lax.rsqrt(deg)
    x = x_ref[...]
    y_ref[0:N, :] = x * dinv[:, None]
    y_ref[N:NPAD, :] = jnp.zeros((NPAD - N, D), jnp.float32)
    a = jnp.sqrt(deg)
    b = jnp.sqrt(jnp.sum(x * x, axis=1))
    ab_ref[...] = jnp.stack([a, b])
    dinv_ref[...] = dinv[:, None]


_tc1 = pl.pallas_call(
    _tc1_body,
    out_shape=(
        jax.ShapeDtypeStruct((NPAD, D), jnp.float32),
        jax.ShapeDtypeStruct((2, N), jnp.float32),
        jax.ShapeDtypeStruct((N, 1), jnp.float32),
    ),
)


@functools.partial(
    pl.kernel,
    mesh=_mesh,
    out_type=(
        jax.ShapeDtypeStruct((NC * AGG, D), jnp.float32),
        jax.ShapeDtypeStruct((NW, 16), jnp.float32),
    ),
    scratch_types=[
        pltpu.VMEM((N,), jnp.float32),
        pltpu.VMEM((N,), jnp.float32),
        pltpu.VMEM((2, 2, B), jnp.int32),
        pltpu.VMEM((2, B), jnp.int32),
        pltpu.VMEM((2, B), jnp.float32),
        pltpu.VMEM((2, B, D), jnp.float32),
        pltpu.VMEM((2, B, D), jnp.float32),
        pltpu.VMEM((16,), jnp.float32),
        pltpu.VMEM_SHARED((AGG, D), jnp.float32),
        pltpu.SemaphoreType.DMA,
        pltpu.SemaphoreType.DMA,
        pltpu.SemaphoreType.DMA,
        pltpu.SemaphoreType.DMA,
        pltpu.SemaphoreType.DMA,
        pltpu.SemaphoreType.DMA,
        pltpu.SemaphoreType.DMA,
        pltpu.SemaphoreType.DMA,
    ],
    compiler_params=pltpu.CompilerParams(needs_layout_passes=False),
)
def _edge_kernel(y_hbm, ei_hbm, ab_hbm, pout_hbm, cos_hbm,
                 a_v, b_v, se_v, dio_v, cw_v, ys_v, yd_v, ca_v,
                 agg_sh, sg0, sg1, sh0, sh1, ss0, ss1, si0, si1):
    cid = lax.axis_index("c")
    sid = lax.axis_index("s")
    wid = cid * NS + sid
    sg = (sg0, sg1)
    sh = (sh0, sh1)
    ss = (ss0, ss1)
    si = (si0, si1)
    pltpu.sync_copy(ab_hbm.at[0], a_v)
    pltpu.sync_copy(ab_hbm.at[1], b_v)

    zeros = jnp.zeros((16,), jnp.float32)

    def zb(i, carry):
        r = i // (D // 16)
        c = i % (D // 16)
        ys_v[0, r, pl.ds(c * 16, 16)] = zeros
        return carry

    lax.fori_loop(0, B * (D // 16), zb, 0)

    rpt = AGG // NS
    zc = 64

    def za(t, carry):
        pltpu.sync_copy(ys_v.at[0, pl.ds(0, zc)],
                        agg_sh.at[pl.ds(sid * rpt + t * zc, zc)])
        return carry

    lax.fori_loop(0, rpt // zc, za, 0)
    plsc.subcore_barrier()

    iota = lax.iota(jnp.int32, 16)
    nbase = cid * NH

    def idx_start(k, slot):
        pltpu.async_copy(ei_hbm.at[k * NS + sid], se_v.at[slot], si[slot])

    def idx_wait(slot):
        pltpu.make_async_copy(ei_hbm.at[sid], se_v.at[slot], si[slot]).wait()

    def gather_start(slot):
        pltpu.async_copy(y_hbm.at[se_v.at[slot, 0]], ys_v.at[slot], sg[slot])
        pltpu.async_copy(y_hbm.at[se_v.at[slot, 1]], yd_v.at[slot], sh[slot])

    def gather_wait(slot):
        pltpu.make_async_copy(
            y_hbm.at[se_v.at[slot, 0]], ys_v.at[slot], sg[slot]).wait()
        pltpu.make_async_copy(
            y_hbm.at[se_v.at[slot, 1]], yd_v.at[slot], sh[slot]).wait()

    def scatter_start(slot):
        pltpu.async_copy(ys_v.at[slot], agg_sh.at[dio_v.at[slot]],
                         ss[slot], add=True)

    def scatter_wait(slot):
        pltpu.make_async_copy(
            ys_v.at[slot], agg_sh.at[dio_v.at[slot]], ss[slot]).wait()

    idx_start(0, 0)
    idx_wait(0)
    gather_start(0)
    idx_start(1, 1)

    dbase = cid * DH

    def pair_body(i, acc):
        for b in range(2):
            k = i * 2 + b
            nxt = 1 - b

            @pl.when(k < KPT - 1)
            def _():
                idx_wait(nxt)

            @pl.when(jnp.logical_and(k > 0, k < KPT - 1))
            def _():
                scatter_wait(nxt)

            @pl.when(k < KPT - 1)
            def _():
                gather_start(nxt)

            for g in range(B // 16):
                sl = pl.ds(g * 16, 16)
                s16 = se_v[b, 0, sl]
                d16 = se_v[b, 1, sl]
                dloc = d16 - nbase
                mine = (dloc >= 0) & (dloc < NH)
                dio_v[b, sl] = jnp.where(mine, dloc, NH)
                a_s = plsc.load_gather(a_v, [s16])
                a_d = plsc.load_gather(a_v, [d16])
                b_s = plsc.load_gather(b_v, [s16])
                b_d = plsc.load_gather(b_v, [d16])
                cw_v[b, sl] = (a_s * a_d) / (b_s * b_d + 1e-8)

            @pl.when(k < KPT - 2)
            def _():
                idx_start(k + 2, b)

            gather_wait(b)
            for g in range(B // 16):
                def edge_body(it, dot16):
                    for u in range(4):
                        e = g * 16 + it * 4 + u
                        ps = None
                        for q in range(DH // 16):
                            dsl = pl.ds(dbase + q * 16, 16)
                            v1 = ys_v[b, e, dsl]
                            v2 = yd_v[b, e, dsl]
                            ps = v1 * v2 if ps is None else ps + v1 * v2
                        s = jnp.sum(ps)
                        dot16 = jnp.where(iota == it * 4 + u, s, dot16)
                    return dot16

                dot = lax.fori_loop(0, 4, edge_body, zeros)
                c16 = cw_v[b, pl.ds(g * 16, 16)]
                eglob = (k * NS + sid) * B + g * 16 + iota
                acc = acc + jnp.where(eglob < E, dot * c16, 0.0)
            scatter_start(b)
        return acc

    acc = lax.fori_loop(0, KPT // 2, pair_body, zeros)
    scatter_wait(0)
    scatter_wait(1)
    ca_v[...] = acc
    pltpu.sync_copy(ca_v, cos_hbm.at[wid])
    plsc.subcore_barrier()

    def wb(t, carry):
        row = sid * rpt + t * zc
        pltpu.sync_copy(agg_sh.at[pl.ds(row, zc)],
                        pout_hbm.at[pl.ds(cid * AGG + row, zc)])
        return carry

    lax.fori_loop(0, rpt // zc, wb, 0)


def _tc2_body(p_ref, x_ref, x0_ref, dinv_ref, w_ref, lw_ref, g_ref, bb_ref,
              cos_ref, out_ref, gcd_ref):
    dinv = dinv_ref[...]
    p = jnp.concatenate(
        [p_ref[0:NH, :], p_ref[AGG:AGG + NH, :]], axis=0)
    x = x_ref[...]
    agg = dinv * p + (dinv * dinv) * x
    h = (1.0 - ALPHA) * agg + ALPHA * x0_ref[...]
    hw = lax.dot_general(h, w_ref[...], (((1,), (1,)), ((), ())),
                         preferred_element_type=jnp.float32)
    out1 = (1.0 - BETA) * h + BETA * hw
    mu = jnp.mean(out1, axis=0)
    cen = out1 - mu[None, :]
    var = jnp.mean(cen * cen, axis=0)
    o = cen * lax.rsqrt(var + 1e-5)[None, :] * g_ref[...] + bb_ref[...]
    o = jnp.maximum(o, 0.0)
    xl = lax.dot_general(x, lw_ref[...], (((1,), (1,)), ((), ())),
                         preferred_element_type=jnp.float32)
    out_ref[...] = o + xl
    gcd_ref[...] = jnp.reshape(1.0 - jnp.sum(cos_ref[...]) * (1.0 / E), (1, 1))


_tc2 = pl.pallas_call(
    _tc2_body,
    out_shape=(
        jax.ShapeDtypeStruct((N, D), jnp.float32),
        jax.ShapeDtypeStruct((1, 1), jnp.float32),
    ),
)


def kernel(x, x_0, edge_index, W, lin_w, bn_gamma, bn_beta):
    src = edge_index[0].astype(jnp.int32)
    dst = edge_index[1].astype(jnp.int32)
    pad = E_PAD - E
    src_p = jnp.concatenate([src, jnp.zeros((pad,), jnp.int32)])
    dst_p = jnp.concatenate(
        [dst, N + (jnp.arange(pad, dtype=jnp.int32) % (NPAD - N))])

    edges_p = jnp.stack(
        [src_p.reshape(NBATCH, B), dst_p.reshape(NBATCH, B)], axis=1)

    deg_parts = _deg_kernel(dst.reshape(NW, EPW))
    y, ab, dinv = _tc1(deg_parts, x)
    pout, cosp = _edge_kernel(y, edges_p, ab)
    out, gcd = _tc2(pout, x, x_0, dinv, W, lin_w,
                    bn_gamma.reshape(1, D), bn_beta.reshape(1, D), cosp)
    return out, gcd.reshape(())
